# baseline (speedup 1.0000x reference)
import functools

import jax
import jax.numpy as jnp
from jax import lax
from jax.experimental import pallas as pl
from jax.experimental.pallas import tpu as pltpu
from jax.experimental.pallas import tpu_sc as plsc

DIM = 128
BATCH = 512
L = 16
NC, NS = 2, 16
NW = NC * NS
RPW = BATCH // NW
NCHUNK = DIM // L

_CBLOCKS = [0, 1, 11, 12, 5, 7, 13, 15, 9, 10, 2, 3, 4]
_RBLOCKS = [6, 14, 8]


def _sc_body(cE, rE, idx_all, out,
             ib, cidx, ridx, gbuf, rbuf, accbuf,
             isem, osem, sems):
    cid = lax.axis_index("c")
    sid = lax.axis_index("s")
    wid = sid * NC + cid
    base = wid * RPW
    iota = lax.iota(jnp.int32, L)
    zero = jnp.zeros((L,), jnp.float32)

    icp = pltpu.make_async_copy(idx_all.at[pl.ds(base, RPW)], ib, isem)
    icp.start()
    icp.wait()

    def col(j):
        return plsc.load_gather(ib, [iota, jnp.full((L,), j, jnp.int32)])

    for b, j in enumerate(_CBLOCKS):
        cidx[pl.ds(b * L, L)] = col(j)
    for b, j in enumerate(_RBLOCKS):
        ridx[pl.ds(b * L, L)] = col(j)

    cps = [
        pltpu.make_async_copy(rE.at[ridx], rbuf, sems.at[0]),
        pltpu.make_async_copy(cE.at[cidx.at[pl.ds(0, 128)]],
                              gbuf.at[pl.ds(0, 128)], sems.at[1]),
        pltpu.make_async_copy(cE.at[cidx.at[pl.ds(128, 80)]],
                              gbuf.at[pl.ds(128, 80)], sems.at[2]),
    ]
    for cp in cps:
        cp.start()
    cps[0].wait()
    cps[1].wait()

    def cc_loss(q, cb, db, rb, r_sign, co_sign):
        def row(i, _):
            def chunk(k, inner):
                accs = []
                for h, acc in enumerate(inner):
                    kk = 2 * k + h
                    c1 = gbuf[cb + i, pl.ds(kk * L, L)]
                    d1 = gbuf[db + i, pl.ds(kk * L, L)]
                    co = jnp.abs(gbuf[cb + i, pl.ds(DIM + kk * L, L)])
                    do = jnp.abs(gbuf[db + i, pl.ds(DIM + kk * L, L)])
                    cen = c1 - d1
                    if rb is not None:
                        r = rbuf[rb + i, pl.ds(kk * L, L)]
                        cen = cen + r if r_sign > 0 else cen - r
                    euc = jnp.abs(cen)
                    if co_sign > 0:
                        t = jnp.maximum(euc + co - do, 0.0)
                    else:
                        t = jnp.maximum(euc - co - do, 0.0)
                    accs.append(acc + t * t)
                return tuple(accs)
            n0, n1 = lax.fori_loop(0, NCHUNK // 2, chunk, (zero, zero))
            accbuf[i, pl.ds(q * L, L)] = n0 + n1
            return 0
        lax.fori_loop(0, RPW, row, 0)

    cc_loss(0, 0 * RPW, 1 * RPW, None, 0, +1)

    def dj_row(i, _):
        def chunk(k, inner):
            accs = []
            for h, acc in enumerate(inner):
                kk = 2 * k + h
                c1 = gbuf[2 * RPW + i, pl.ds(kk * L, L)]
                d1 = gbuf[3 * RPW + i, pl.ds(kk * L, L)]
                co = jnp.abs(gbuf[2 * RPW + i, pl.ds(DIM + kk * L, L)])
                do = jnp.abs(gbuf[3 * RPW + i, pl.ds(DIM + kk * L, L)])
                t = jnp.maximum(co + do - jnp.abs(c1 - d1), 0.0)
                accs.append(acc + t * t)
            return tuple(accs)
        n0, n1 = lax.fori_loop(0, NCHUNK // 2, chunk, (zero, zero))
        accbuf[i, pl.ds(1 * L, L)] = n0 + n1
        return 0
    lax.fori_loop(0, RPW, dj_row, 0)

    cc_loss(2, 4 * RPW, 5 * RPW, 0 * RPW, +1, +1)
    cc_loss(3, 6 * RPW, 7 * RPW, 1 * RPW, +1, -1)
    cps[2].wait()
    cc_loss(4, 8 * RPW, 9 * RPW, 2 * RPW, -1, -1)

    C2, D2, E2 = 10 * RPW, 11 * RPW, 12 * RPW

    def nf2_row(i, _):
        def chunk(k, carry):
            aa, bb = carry
            c1 = gbuf[C2 + i, pl.ds(k * L, L)]
            d1 = gbuf[D2 + i, pl.ds(k * L, L)]
            e1 = gbuf[E2 + i, pl.ds(k * L, L)]
            c2 = jnp.abs(gbuf[C2 + i, pl.ds(DIM + k * L, L)])
            d2 = jnp.abs(gbuf[D2 + i, pl.ds(DIM + k * L, L)])
            e2 = jnp.abs(gbuf[E2 + i, pl.ds(DIM + k * L, L)])
            start = jnp.maximum(c1 - c2, d1 - d2)
            end = jnp.minimum(c1 + c2, d1 + d2)
            diff = start - end
            new_r = jnp.abs(diff) * 0.5
            cen1 = (start + end) * 0.5
            u = jnp.maximum(jnp.abs(cen1 - e1) + new_r - e2, 0.0)
            v = jnp.maximum(diff, 0.0)
            return aa + u * u, bb + v * v
        aa, bb = lax.fori_loop(0, NCHUNK, chunk, (zero, zero))
        accbuf[i, pl.ds(5 * L, L)] = aa
        accbuf[i, pl.ds(6 * L, L)] = bb
        accbuf[i, pl.ds(7 * L, L)] = zero
        return 0
    lax.fori_loop(0, RPW, nf2_row, 0)

    ocp = pltpu.make_async_copy(accbuf, out.at[pl.ds(base, RPW)], osem)
    ocp.start()
    ocp.wait()


@functools.cache
def _make_sc_kernel():
    return pl.kernel(
        _sc_body,
        out_type=jax.ShapeDtypeStruct((BATCH, 2 * DIM), jnp.float32),
        mesh=plsc.VectorSubcoreMesh(core_axis_name="c", subcore_axis_name="s"),
        compiler_params=pltpu.CompilerParams(needs_layout_passes=False),
        scratch_types=[
            pltpu.VMEM((RPW, 16), jnp.int32),
            pltpu.VMEM((13 * RPW,), jnp.int32),
            pltpu.VMEM((3 * RPW,), jnp.int32),
            pltpu.VMEM((13 * RPW, 2 * DIM), jnp.float32),
            pltpu.VMEM((3 * RPW, DIM), jnp.float32),
            pltpu.VMEM((RPW, 2 * DIM), jnp.float32),
            pltpu.SemaphoreType.DMA,
            pltpu.SemaphoreType.DMA,
            pltpu.SemaphoreType.DMA((3,)),
        ],
    )


def _finish_body(p, out):
    x = p[...]
    inv_b = 1.0 / BATCH
    blk = [x[:, q * L:(q + 1) * L] for q in range(7)]
    loss1 = jnp.sum(blk[0]) * inv_b
    dj = jnp.sum(blk[1]) * inv_b
    loss3 = jnp.sum(blk[2]) * inv_b
    loss4 = jnp.sum(blk[4]) * inv_b
    a2 = jnp.sum(blk[5], axis=1, keepdims=True)
    b2 = jnp.sum(blk[6], axis=1, keepdims=True)
    mean_a = jnp.sum(jnp.sqrt(a2)) * inv_b
    mean_b = jnp.sum(jnp.sqrt(b2)) * inv_b
    loss2 = (jnp.sum(a2) + jnp.sum(b2)) * inv_b + 2.0 * mean_a * mean_b
    n2 = jnp.sum(blk[3], axis=1, keepdims=True)
    dn = jnp.sqrt(n2)
    neg = jnp.sum((dn - 2.0) ** 2) * inv_b
    out[0, 0] = loss1 + loss2 + dj + loss3 + loss4 + neg


_finish = pl.pallas_call(
    _finish_body,
    out_shape=jax.ShapeDtypeStruct((1, 1), jnp.float32),
    out_specs=pl.BlockSpec(memory_space=pltpu.SMEM),
)


def kernel(classEmb, relEmb, nf1, nf2, nf3, nf4, disjoint, nf3_neg):
    idx_all = jnp.concatenate(
        [nf1[:BATCH], nf2[:BATCH], nf3[:BATCH], nf4[:BATCH],
         disjoint[:BATCH], nf3_neg[:BATCH]], axis=1)
    parts = _make_sc_kernel()(classEmb, relEmb, idx_all)
    return _finish(parts).reshape(())


# speedup vs baseline: 1.0256x; 1.0256x over previous
import functools

import jax
import jax.numpy as jnp
from jax import lax
from jax.experimental import pallas as pl
from jax.experimental.pallas import tpu as pltpu
from jax.experimental.pallas import tpu_sc as plsc

DIM = 128
BATCH = 512
L = 16
NC, NS = 2, 16
NW = NC * NS
RPW = BATCH // NW
NCHUNK = DIM // L

_CBLOCKS = [0, 1, 11, 12, 5, 7, 13, 15, 9, 10, 2, 3, 4]
_RBLOCKS = [6, 14, 8]


def _sc_body(cE, rE, idx_all, out,
             ib, cidx, ridx, gbuf, rbuf, accbuf,
             isem, osem, sems):
    cid = lax.axis_index("c")
    sid = lax.axis_index("s")
    wid = sid * NC + cid
    base = wid * RPW
    iota = lax.iota(jnp.int32, L)
    zero = jnp.zeros((L,), jnp.float32)

    icp = pltpu.make_async_copy(idx_all.at[pl.ds(base, RPW)], ib, isem)
    icp.start()
    icp.wait()

    def col(j):
        return plsc.load_gather(ib, [iota, jnp.full((L,), j, jnp.int32)])

    for b, j in enumerate(_CBLOCKS):
        cidx[pl.ds(b * L, L)] = col(j)
    for b, j in enumerate(_RBLOCKS):
        ridx[pl.ds(b * L, L)] = col(j)

    cps = [pltpu.make_async_copy(rE.at[ridx], rbuf, sems.at[0])]
    for s in range(5):
        cps.append(pltpu.make_async_copy(
            cE.at[cidx.at[pl.ds(s * 32, 32)]],
            gbuf.at[pl.ds(s * 32, 32)], sems.at[s + 1]))
    cps.append(pltpu.make_async_copy(
        cE.at[cidx.at[pl.ds(160, 48)]],
        gbuf.at[pl.ds(160, 48)], sems.at[6]))
    for cp in cps:
        cp.start()
    cps[0].wait()
    cps[1].wait()

    def cc_loss(q, cb, db, rb, r_sign, co_sign):
        def row(i, _):
            def chunk(k, inner):
                accs = []
                for h, acc in enumerate(inner):
                    kk = 2 * k + h
                    c1 = gbuf[cb + i, pl.ds(kk * L, L)]
                    d1 = gbuf[db + i, pl.ds(kk * L, L)]
                    co = jnp.abs(gbuf[cb + i, pl.ds(DIM + kk * L, L)])
                    do = jnp.abs(gbuf[db + i, pl.ds(DIM + kk * L, L)])
                    cen = c1 - d1
                    if rb is not None:
                        r = rbuf[rb + i, pl.ds(kk * L, L)]
                        cen = cen + r if r_sign > 0 else cen - r
                    euc = jnp.abs(cen)
                    if co_sign > 0:
                        t = jnp.maximum(euc + co - do, 0.0)
                    else:
                        t = jnp.maximum(euc - co - do, 0.0)
                    accs.append(acc + t * t)
                return tuple(accs)
            n0, n1 = lax.fori_loop(0, NCHUNK // 2, chunk, (zero, zero))
            accbuf[i, pl.ds(q * L, L)] = n0 + n1
            return 0
        lax.fori_loop(0, RPW, row, 0)

    cc_loss(0, 0 * RPW, 1 * RPW, None, 0, +1)
    cps[2].wait()

    def dj_row(i, _):
        def chunk(k, inner):
            accs = []
            for h, acc in enumerate(inner):
                kk = 2 * k + h
                c1 = gbuf[2 * RPW + i, pl.ds(kk * L, L)]
                d1 = gbuf[3 * RPW + i, pl.ds(kk * L, L)]
                co = jnp.abs(gbuf[2 * RPW + i, pl.ds(DIM + kk * L, L)])
                do = jnp.abs(gbuf[3 * RPW + i, pl.ds(DIM + kk * L, L)])
                t = jnp.maximum(co + do - jnp.abs(c1 - d1), 0.0)
                accs.append(acc + t * t)
            return tuple(accs)
        n0, n1 = lax.fori_loop(0, NCHUNK // 2, chunk, (zero, zero))
        accbuf[i, pl.ds(1 * L, L)] = n0 + n1
        return 0
    lax.fori_loop(0, RPW, dj_row, 0)

    cps[3].wait()
    cc_loss(2, 4 * RPW, 5 * RPW, 0 * RPW, +1, +1)
    cps[4].wait()
    cc_loss(3, 6 * RPW, 7 * RPW, 1 * RPW, +1, -1)
    cps[5].wait()
    cc_loss(4, 8 * RPW, 9 * RPW, 2 * RPW, -1, -1)
    cps[6].wait()

    C2, D2, E2 = 10 * RPW, 11 * RPW, 12 * RPW

    def nf2_row(i, _):
        def chunk(k, carry):
            aa, bb = carry
            c1 = gbuf[C2 + i, pl.ds(k * L, L)]
            d1 = gbuf[D2 + i, pl.ds(k * L, L)]
            e1 = gbuf[E2 + i, pl.ds(k * L, L)]
            c2 = jnp.abs(gbuf[C2 + i, pl.ds(DIM + k * L, L)])
            d2 = jnp.abs(gbuf[D2 + i, pl.ds(DIM + k * L, L)])
            e2 = jnp.abs(gbuf[E2 + i, pl.ds(DIM + k * L, L)])
            start = jnp.maximum(c1 - c2, d1 - d2)
            end = jnp.minimum(c1 + c2, d1 + d2)
            diff = start - end
            new_r = jnp.abs(diff) * 0.5
            cen1 = (start + end) * 0.5
            u = jnp.maximum(jnp.abs(cen1 - e1) + new_r - e2, 0.0)
            v = jnp.maximum(diff, 0.0)
            return aa + u * u, bb + v * v
        aa, bb = lax.fori_loop(0, NCHUNK, chunk, (zero, zero))
        accbuf[i, pl.ds(5 * L, L)] = aa
        accbuf[i, pl.ds(6 * L, L)] = bb
        accbuf[i, pl.ds(7 * L, L)] = zero
        return 0
    lax.fori_loop(0, RPW, nf2_row, 0)

    ocp = pltpu.make_async_copy(accbuf, out.at[pl.ds(base, RPW)], osem)
    ocp.start()
    ocp.wait()


@functools.cache
def _make_sc_kernel():
    return pl.kernel(
        _sc_body,
        out_type=jax.ShapeDtypeStruct((BATCH, 2 * DIM), jnp.float32),
        mesh=plsc.VectorSubcoreMesh(core_axis_name="c", subcore_axis_name="s"),
        compiler_params=pltpu.CompilerParams(needs_layout_passes=False),
        scratch_types=[
            pltpu.VMEM((RPW, 16), jnp.int32),
            pltpu.VMEM((13 * RPW,), jnp.int32),
            pltpu.VMEM((3 * RPW,), jnp.int32),
            pltpu.VMEM((13 * RPW, 2 * DIM), jnp.float32),
            pltpu.VMEM((3 * RPW, DIM), jnp.float32),
            pltpu.VMEM((RPW, 2 * DIM), jnp.float32),
            pltpu.SemaphoreType.DMA,
            pltpu.SemaphoreType.DMA,
            pltpu.SemaphoreType.DMA((7,)),
        ],
    )


def _finish_body(p, out):
    x = p[...]
    inv_b = 1.0 / BATCH
    blk = [x[:, q * L:(q + 1) * L] for q in range(7)]
    loss1 = jnp.sum(blk[0]) * inv_b
    dj = jnp.sum(blk[1]) * inv_b
    loss3 = jnp.sum(blk[2]) * inv_b
    loss4 = jnp.sum(blk[4]) * inv_b
    a2 = jnp.sum(blk[5], axis=1, keepdims=True)
    b2 = jnp.sum(blk[6], axis=1, keepdims=True)
    mean_a = jnp.sum(jnp.sqrt(a2)) * inv_b
    mean_b = jnp.sum(jnp.sqrt(b2)) * inv_b
    loss2 = (jnp.sum(a2) + jnp.sum(b2)) * inv_b + 2.0 * mean_a * mean_b
    n2 = jnp.sum(blk[3], axis=1, keepdims=True)
    dn = jnp.sqrt(n2)
    neg = jnp.sum((dn - 2.0) ** 2) * inv_b
    out[0, 0] = loss1 + loss2 + dj + loss3 + loss4 + neg


_finish = pl.pallas_call(
    _finish_body,
    out_shape=jax.ShapeDtypeStruct((1, 1), jnp.float32),
    out_specs=pl.BlockSpec(memory_space=pltpu.SMEM),
)


def kernel(classEmb, relEmb, nf1, nf2, nf3, nf4, disjoint, nf3_neg):
    idx_all = jnp.concatenate(
        [nf1[:BATCH], nf2[:BATCH], nf3[:BATCH], nf4[:BATCH],
         disjoint[:BATCH], nf3_neg[:BATCH]], axis=1)
    parts = _make_sc_kernel()(classEmb, relEmb, idx_all)
    return _finish(parts).reshape(())
